# baseline (speedup 1.0000x reference)
import functools

import jax
import jax.numpy as jnp
from jax import lax
from jax.experimental import pallas as pl
from jax.experimental.pallas import tpu as pltpu
from jax.experimental.pallas import tpu_sc as plsc

NC = 2
NS = 16
NW = NC * NS
LANES = 16
CHB = 896
GG = 112
NCHUNK = 14
PW = CHB * NCHUNK


def _mesh():
    return plsc.VectorSubcoreMesh(
        core_axis_name="c", subcore_axis_name="s", num_cores=NC,
        num_subcores=NS)


def _params():
    return pltpu.CompilerParams(
        use_tc_tiling_on_sc=False, needs_layout_passes=False)


def _make_table_builder(NP, RW):
    SB = RW // 2

    @functools.partial(
        pl.kernel,
        out_type=jax.ShapeDtypeStruct((2 * NP, LANES), jnp.float32),
        mesh=_mesh(),
        scratch_types=[
            pltpu.VMEM((SB, LANES), jnp.float32),
            pltpu.VMEM((SB, LANES), jnp.float32),
        ],
        compiler_params=_params(),
    )
    def build(vt_hbm, tbl_hbm, vbuf, tbuf):
        wid = lax.axis_index("c") * NS + lax.axis_index("s")
        r0 = wid * RW
        for h in range(2):
            base = r0 + h * SB
            pltpu.sync_copy(vt_hbm.at[pl.ds(base, SB)], vbuf)

            def pa(i, carry):
                r = i * 4
                for u in range(4):
                    tbuf[r + u] = 0.5 - 0.5 * vbuf[r + u]
                return carry

            lax.fori_loop(0, SB // 4, pa, 0)
            pltpu.sync_copy(tbuf, tbl_hbm.at[pl.ds(base, SB)])

            def pb(i, carry):
                r = i * 4
                for u in range(4):
                    tbuf[r + u] = 0.5 + 0.5 * vbuf[r + u]
                return carry

            lax.fori_loop(0, SB // 4, pb, 0)
            pltpu.sync_copy(tbuf, tbl_hbm.at[pl.ds(NP + base, SB)])

    return build


def _make_main(NP, M, n_tiles, n_extra, n_part):
    assert n_part % LANES == 0

    @functools.partial(
        pl.kernel,
        out_type=jax.ShapeDtypeStruct((n_tiles * LANES, CHB), jnp.float32),
        mesh=_mesh(),
        scratch_types=[
            pltpu.VMEM((2, 3, CHB), jnp.int32),
            pltpu.VMEM((2, 3, CHB), jnp.float32),
            pltpu.VMEM((2, 3, CHB, LANES), jnp.float32),
            pltpu.VMEM((2, LANES, CHB), jnp.float32),
            pltpu.SemaphoreType.DMA,
            pltpu.SemaphoreType.DMA,
            pltpu.SemaphoreType.DMA,
            pltpu.SemaphoreType.DMA,
        ],
        compiler_params=_params(),
    )
    def main(tbl, i0, i1, i2, s0, s1, s2, out, idxv, sgnv, gbuf, obuf,
             gsem0, gsem1, osem0, osem1):
        gsem = (gsem0, gsem1)
        osem = (osem0, osem1)
        irefs = (i0, i1, i2)
        srefs = (s0, s1, s2)
        wid = lax.axis_index("c") * NS + lax.axis_index("s")
        t0 = wid * NCHUNK
        iota = lax.iota(jnp.int32, LANES)

        def adjust(p, n):
            def abody(g, carry):
                o = g * 64
                for k in range(3):
                    for u in range(4):
                        oo = o + u * LANES
                        ii = idxv[p, k, pl.ds(oo, LANES)]
                        ss = sgnv[p, k, pl.ds(oo, LANES)]
                        idxv[p, k, pl.ds(oo, LANES)] = ii + jnp.where(
                            ss < 0.0, jnp.int32(NP), jnp.int32(0))
                return carry

            lax.fori_loop(0, n // 64, abody, 0)

        def load_fire(t, p):
            base = t * CHB
            for k in range(3):
                pltpu.sync_copy(irefs[k].at[pl.ds(base, CHB)], idxv.at[p, k])
                pltpu.sync_copy(srefs[k].at[pl.ds(base, CHB)], sgnv.at[p, k])
            adjust(p, CHB)
            for k in range(3):
                for j in range(CHB // GG):
                    pltpu.async_copy(
                        tbl.at[idxv.at[p, k, pl.ds(j * GG, GG)]],
                        gbuf.at[p, k, pl.ds(j * GG, GG)],
                        gsem[p])

        def wait_gather(p):
            for k in range(3):
                pltpu.make_async_copy(
                    tbl.at[pl.ds(0, CHB)], gbuf.at[p, k], gsem[p]).wait()

        def compute(p, n):
            def cbody(i, carry):
                c = i * 4
                for u in range(4):
                    m = jnp.minimum(
                        jnp.minimum(gbuf[p, 0, c + u], gbuf[p, 1, c + u]),
                        gbuf[p, 2, c + u])
                    plsc.store_scatter(
                        obuf.at[p], [iota, iota * 0 + (c + u)], m)
                return carry

            lax.fori_loop(0, n // 4, cbody, 0)

        def flush_out(t, p):
            pltpu.async_copy(
                obuf.at[p], out.at[pl.ds(t * LANES, LANES)], osem[p])

        def wait_out(p):
            pltpu.make_async_copy(
                obuf.at[p], out.at[pl.ds(0, LANES)], osem[p]).wait()

        def step(ci, p, do_wait_out, next_ci):
            wait_gather(p)
            if do_wait_out:
                wait_out(p)
            compute(p, CHB)
            flush_out(t0 + ci, p)
            if next_ci is not None:
                load_fire(t0 + next_ci, p)

        load_fire(t0, 0)
        load_fire(t0 + 1, 1)
        step(0, 0, False, 2)
        step(1, 1, False, 3)

        def pair(t, carry):
            ca = 2 * t
            step(ca, 0, True, ca + 2)
            step(ca + 1, 1, True, ca + 3)
            return carry

        lax.fori_loop(1, NCHUNK // 2 - 1, pair, 0)
        step(NCHUNK - 2, 0, True, None)
        step(NCHUNK - 1, 1, True, None)
        wait_out(0)
        wait_out(1)

        @pl.when(wid < n_extra)
        def _():
            te = NW * NCHUNK + wid
            load_fire(te, 0)
            wait_gather(0)
            compute(0, CHB)
            flush_out(te, 0)
            wait_out(0)

        if n_part:
            @pl.when(wid == n_extra)
            def _():
                tp = NW * NCHUNK + n_extra
                base = tp * CHB
                gsz = [GG] * (n_part // GG)
                if n_part % GG:
                    gsz.append(n_part % GG)
                for k in range(3):
                    pltpu.sync_copy(irefs[k].at[pl.ds(base, n_part)],
                                    idxv.at[0, k, pl.ds(0, n_part)])
                    pltpu.sync_copy(srefs[k].at[pl.ds(base, n_part)],
                                    sgnv.at[0, k, pl.ds(0, n_part)])

                def abody(g, carry):
                    o = g * LANES
                    for k in range(3):
                        ii = idxv[0, k, pl.ds(o, LANES)]
                        ss = sgnv[0, k, pl.ds(o, LANES)]
                        idxv[0, k, pl.ds(o, LANES)] = ii + jnp.where(
                            ss < 0.0, jnp.int32(NP), jnp.int32(0))
                    return carry

                lax.fori_loop(0, n_part // LANES, abody, 0)
                for k in range(3):
                    o = 0
                    for g in gsz:
                        pltpu.async_copy(
                            tbl.at[idxv.at[0, k, pl.ds(o, g)]],
                            gbuf.at[0, k, pl.ds(o, g)], gsem0)
                        o += g
                for k in range(3):
                    pltpu.make_async_copy(
                        tbl.at[pl.ds(0, n_part)],
                        gbuf.at[0, k, pl.ds(0, n_part)], gsem0).wait()

                def cbody(i, carry):
                    m = jnp.minimum(
                        jnp.minimum(gbuf[0, 0, i], gbuf[0, 1, i]),
                        gbuf[0, 2, i])
                    plsc.store_scatter(obuf.at[0], [iota, iota * 0 + i], m)
                    return carry

                lax.fori_loop(0, n_part, cbody, 0)
                pltpu.async_copy(
                    obuf.at[0], out.at[pl.ds(tp * LANES, LANES)], osem0)
                wait_out(0)

    return main


def _make_remap(M, n_tiles):
    def body(x_ref, o_ref):
        o_ref[...] = x_ref[...]

    return pl.pallas_call(
        body,
        grid=(n_tiles,),
        in_specs=[pl.BlockSpec((LANES, CHB), lambda i: (i, 0))],
        out_specs=pl.BlockSpec((LANES, CHB), lambda i: (0, i)),
        out_shape=jax.ShapeDtypeStruct((LANES, M), jnp.float32),
    )


def kernel(v, input_idx, input_sign):
    B, N = v.shape
    M, K = input_idx.shape
    assert B == LANES and K == 3

    NP = (N + NW * 8 - 1) // (NW * 8) * (NW * 8)
    RW = NP // NW

    n_full = M // CHB
    n_part = M - n_full * CHB
    n_extra = n_full - NW * NCHUNK
    assert 0 <= n_extra < NW
    n_tiles = n_full + (1 if n_part else 0)

    vt = jnp.zeros((NP, LANES), jnp.float32).at[:N].set(v.T)
    tbl = _make_table_builder(NP, RW)(vt)
    stack = _make_main(NP, M, n_tiles, n_extra, n_part)(
        tbl,
        input_idx[:, 0], input_idx[:, 1], input_idx[:, 2],
        input_sign[:, 0], input_sign[:, 1], input_sign[:, 2])
    return _make_remap(M, n_tiles)(stack)


# speedup vs baseline: 1.4575x; 1.4575x over previous
import functools

import jax
import jax.numpy as jnp
from jax import lax
from jax.experimental import pallas as pl
from jax.experimental.pallas import tpu as pltpu
from jax.experimental.pallas import tpu_sc as plsc

NC = 2
NS = 16
NW = NC * NS
LANES = 16
CH = 832
CH3 = CH * 3
GG = 104
NCHUNK = 16
PW = CH * NCHUNK


def _mesh():
    return plsc.VectorSubcoreMesh(
        core_axis_name="c", subcore_axis_name="s", num_cores=NC,
        num_subcores=NS)


def _params():
    return pltpu.CompilerParams(
        use_tc_tiling_on_sc=False, needs_layout_passes=False)


def _make_table_builder(NP, RW):
    SB = RW // 2

    @functools.partial(
        pl.kernel,
        out_type=jax.ShapeDtypeStruct((2 * NP, LANES), jnp.float32),
        mesh=_mesh(),
        scratch_types=[
            pltpu.VMEM((SB, LANES), jnp.float32),
            pltpu.VMEM((SB, LANES), jnp.float32),
        ],
        compiler_params=_params(),
    )
    def build(vt_hbm, tbl_hbm, vbuf, tbuf):
        wid = lax.axis_index("c") * NS + lax.axis_index("s")
        r0 = wid * RW
        for h in range(2):
            base = r0 + h * SB
            pltpu.sync_copy(vt_hbm.at[pl.ds(base, SB)], vbuf)

            def pa(i, carry):
                r = i * 4
                for u in range(4):
                    tbuf[r + u] = 0.5 - 0.5 * vbuf[r + u]
                return carry

            lax.fori_loop(0, SB // 4, pa, 0)
            pltpu.sync_copy(tbuf, tbl_hbm.at[pl.ds(base, SB)])

            def pb(i, carry):
                r = i * 4
                for u in range(4):
                    tbuf[r + u] = 0.5 + 0.5 * vbuf[r + u]
                return carry

            lax.fori_loop(0, SB // 4, pb, 0)
            pltpu.sync_copy(tbuf, tbl_hbm.at[pl.ds(NP + base, SB)])

    return build


def _make_main(NP, M):
    n_tail = M - NW * PW
    assert 0 <= n_tail <= CH and n_tail % LANES == 0

    @functools.partial(
        pl.kernel,
        out_type=jax.ShapeDtypeStruct((M, LANES), jnp.float32),
        mesh=_mesh(),
        scratch_types=[
            pltpu.VMEM((2, 3, CH), jnp.int32),
            pltpu.VMEM((2, 3, CH), jnp.float32),
            pltpu.VMEM((2, 3, CH, LANES), jnp.float32),
            pltpu.VMEM((2, CH, LANES), jnp.float32),
            pltpu.SemaphoreType.DMA,
            pltpu.SemaphoreType.DMA,
            pltpu.SemaphoreType.DMA,
            pltpu.SemaphoreType.DMA,
        ],
        compiler_params=_params(),
    )
    def main(tbl, i0, i1, i2, s0, s1, s2, out, idxv, sgnv, gbuf, obuf,
             gsem0, gsem1, osem0, osem1):
        gsem = (gsem0, gsem1)
        osem = (osem0, osem1)
        irefs = (i0, i1, i2)
        srefs = (s0, s1, s2)
        wid = lax.axis_index("c") * NS + lax.axis_index("s")
        wbase = wid * PW

        def load_fire(ci, p):
            base = wbase + ci * CH
            for k in range(3):
                pltpu.sync_copy(irefs[k].at[pl.ds(base, CH)], idxv.at[p, k])
                pltpu.sync_copy(srefs[k].at[pl.ds(base, CH)], sgnv.at[p, k])

            def abody(g, carry):
                o = g * 64
                for k in range(3):
                    for u in range(4):
                        oo = o + u * LANES
                        ii = idxv[p, k, pl.ds(oo, LANES)]
                        ss = sgnv[p, k, pl.ds(oo, LANES)]
                        idxv[p, k, pl.ds(oo, LANES)] = ii + jnp.where(
                            ss < 0.0, jnp.int32(NP), jnp.int32(0))
                return carry

            lax.fori_loop(0, CH // 64, abody, 0)
            for k in range(3):
                for j in range(CH // GG):
                    pltpu.async_copy(
                        tbl.at[idxv.at[p, k, pl.ds(j * GG, GG)]],
                        gbuf.at[p, k, pl.ds(j * GG, GG)],
                        gsem[p])

        def wait_gather(p):
            for k in range(3):
                pltpu.make_async_copy(
                    tbl.at[pl.ds(0, CH)], gbuf.at[p, k], gsem[p]).wait()

        def compute(p):
            def cbody(i, carry):
                c = i * 8
                for u in range(8):
                    obuf[p, c + u] = jnp.minimum(
                        jnp.minimum(gbuf[p, 0, c + u], gbuf[p, 1, c + u]),
                        gbuf[p, 2, c + u])
                return carry

            lax.fori_loop(0, CH // 8, cbody, 0)

        def flush_out(ci, p):
            pltpu.async_copy(
                obuf.at[p], out.at[pl.ds(wbase + ci * CH, CH)], osem[p])

        def wait_out(p):
            pltpu.make_async_copy(
                obuf.at[p], out.at[pl.ds(0, CH)], osem[p]).wait()

        def step(ci, p, do_wait_out, next_ci):
            wait_gather(p)
            if do_wait_out:
                wait_out(p)
            compute(p)
            flush_out(ci, p)
            if next_ci is not None:
                load_fire(next_ci, p)

        load_fire(0, 0)
        load_fire(1, 1)
        step(0, 0, False, 2)
        step(1, 1, False, 3)

        def pair(t, carry):
            ca = 2 * t
            step(ca, 0, True, ca + 2)
            step(ca + 1, 1, True, ca + 3)
            return carry

        lax.fori_loop(1, NCHUNK // 2 - 1, pair, 0)
        step(NCHUNK - 2, 0, True, None)
        step(NCHUNK - 1, 1, True, None)
        wait_out(0)
        wait_out(1)

        if n_tail:
            @pl.when(wid == 0)
            def _():
                base = NW * PW
                for k in range(3):
                    pltpu.sync_copy(irefs[k].at[pl.ds(base, n_tail)],
                                    idxv.at[0, k, pl.ds(0, n_tail)])
                    pltpu.sync_copy(srefs[k].at[pl.ds(base, n_tail)],
                                    sgnv.at[0, k, pl.ds(0, n_tail)])

                def abody(g, carry):
                    o = g * LANES
                    for k in range(3):
                        ii = idxv[0, k, pl.ds(o, LANES)]
                        ss = sgnv[0, k, pl.ds(o, LANES)]
                        idxv[0, k, pl.ds(o, LANES)] = ii + jnp.where(
                            ss < 0.0, jnp.int32(NP), jnp.int32(0))
                    return carry

                lax.fori_loop(0, n_tail // LANES, abody, 0)
                for k in range(3):
                    pltpu.async_copy(
                        tbl.at[idxv.at[0, k, pl.ds(0, n_tail)]],
                        gbuf.at[0, k, pl.ds(0, n_tail)], gsem0)
                for k in range(3):
                    pltpu.make_async_copy(
                        tbl.at[pl.ds(0, n_tail)],
                        gbuf.at[0, k, pl.ds(0, n_tail)], gsem0).wait()

                def cbody(i, carry):
                    obuf[0, i] = jnp.minimum(
                        jnp.minimum(gbuf[0, 0, i], gbuf[0, 1, i]),
                        gbuf[0, 2, i])
                    return carry

                lax.fori_loop(0, n_tail, cbody, 0)
                pltpu.async_copy(
                    obuf.at[0, pl.ds(0, n_tail)],
                    out.at[pl.ds(base, n_tail)], osem0)
                pltpu.make_async_copy(
                    obuf.at[0, pl.ds(0, n_tail)],
                    out.at[pl.ds(base, n_tail)], osem0).wait()

    return main


def kernel(v, input_idx, input_sign):
    B, N = v.shape
    M, K = input_idx.shape
    assert B == LANES and K == 3

    NP = (N + NW * 8 - 1) // (NW * 8) * (NW * 8)
    RW = NP // NW

    vt = jnp.zeros((NP, LANES), jnp.float32).at[:N].set(v.T)
    tbl = _make_table_builder(NP, RW)(vt)
    outT = _make_main(NP, M)(
        tbl,
        input_idx[:, 0], input_idx[:, 1], input_idx[:, 2],
        input_sign[:, 0], input_sign[:, 1], input_sign[:, 2])
    return outT.T


# speedup vs baseline: 1.8514x; 1.2703x over previous
import functools

import jax
import jax.numpy as jnp
from jax import lax
from jax.experimental import pallas as pl
from jax.experimental.pallas import tpu as pltpu
from jax.experimental.pallas import tpu_sc as plsc

NC = 2
NS = 16
NW = NC * NS
LANES = 16
CHB = 896
CHP = CHB + 1
GG = 112
NCHUNK = 14
KREMAP = 14


def _mesh():
    return plsc.VectorSubcoreMesh(
        core_axis_name="c", subcore_axis_name="s", num_cores=NC,
        num_subcores=NS)


def _params():
    return pltpu.CompilerParams(
        use_tc_tiling_on_sc=False, needs_layout_passes=False)


def _make_table_builder(NP, RW):
    SB = RW // 2

    @functools.partial(
        pl.kernel,
        out_type=jax.ShapeDtypeStruct((2 * NP, LANES), jnp.float32),
        mesh=_mesh(),
        scratch_types=[
            pltpu.VMEM((SB, LANES), jnp.float32),
            pltpu.VMEM((SB, LANES), jnp.float32),
        ],
        compiler_params=_params(),
    )
    def build(vt_hbm, tbl_hbm, vbuf, tbuf):
        wid = lax.axis_index("c") * NS + lax.axis_index("s")
        r0 = wid * RW
        for h in range(2):
            base = r0 + h * SB
            pltpu.sync_copy(vt_hbm.at[pl.ds(base, SB)], vbuf)

            def pa(i, carry):
                r = i * 4
                for u in range(4):
                    tbuf[r + u] = 0.5 - 0.5 * vbuf[r + u]
                return carry

            lax.fori_loop(0, SB // 4, pa, 0)
            pltpu.sync_copy(tbuf, tbl_hbm.at[pl.ds(base, SB)])

            def pb(i, carry):
                r = i * 4
                for u in range(4):
                    tbuf[r + u] = 0.5 + 0.5 * vbuf[r + u]
                return carry

            lax.fori_loop(0, SB // 4, pb, 0)
            pltpu.sync_copy(tbuf, tbl_hbm.at[pl.ds(NP + base, SB)])

    return build


def _make_main(NP, M, n_tiles, n_extra, n_part):
    assert n_part % LANES == 0

    @functools.partial(
        pl.kernel,
        out_type=jax.ShapeDtypeStruct((n_tiles * LANES, CHB), jnp.float32),
        mesh=_mesh(),
        scratch_types=[
            pltpu.VMEM((2, 3, CHB), jnp.int32),
            pltpu.VMEM((2, 3, CHB), jnp.float32),
            pltpu.VMEM((2, 3, CHB, LANES), jnp.float32),
            pltpu.VMEM((2, LANES, CHP), jnp.float32),
            pltpu.SemaphoreType.DMA,
            pltpu.SemaphoreType.DMA,
            pltpu.SemaphoreType.DMA,
            pltpu.SemaphoreType.DMA,
        ],
        compiler_params=_params(),
    )
    def main(tbl, i0, i1, i2, s0, s1, s2, out, idxv, sgnv, gbuf, obuf,
             gsem0, gsem1, osem0, osem1):
        gsem = (gsem0, gsem1)
        osem = (osem0, osem1)
        irefs = (i0, i1, i2)
        srefs = (s0, s1, s2)
        wid = lax.axis_index("c") * NS + lax.axis_index("s")
        t0 = wid * NCHUNK
        iota = lax.iota(jnp.int32, LANES)
        scat_rows = iota * 0 + iota

        def adjust(p, n):
            def abody(g, carry):
                o = g * 64
                for k in range(3):
                    for u in range(4):
                        oo = o + u * LANES
                        ii = idxv[p, k, pl.ds(oo, LANES)]
                        ss = sgnv[p, k, pl.ds(oo, LANES)]
                        idxv[p, k, pl.ds(oo, LANES)] = ii + jnp.where(
                            ss < 0.0, jnp.int32(NP), jnp.int32(0))
                return carry

            lax.fori_loop(0, n // 64, abody, 0)

        def load_fire(t, p):
            base = t * CHB
            for k in range(3):
                pltpu.sync_copy(irefs[k].at[pl.ds(base, CHB)], idxv.at[p, k])
                pltpu.sync_copy(srefs[k].at[pl.ds(base, CHB)], sgnv.at[p, k])
            adjust(p, CHB)
            for k in range(3):
                for j in range(CHB // GG):
                    pltpu.async_copy(
                        tbl.at[idxv.at[p, k, pl.ds(j * GG, GG)]],
                        gbuf.at[p, k, pl.ds(j * GG, GG)],
                        gsem[p])

        def wait_gather(p):
            for k in range(3):
                pltpu.make_async_copy(
                    tbl.at[pl.ds(0, CHB)], gbuf.at[p, k], gsem[p]).wait()

        def compute(p, n):
            def cbody(i, carry):
                c = i * 4
                for u in range(4):
                    m = jnp.minimum(
                        jnp.minimum(gbuf[p, 0, c + u], gbuf[p, 1, c + u]),
                        gbuf[p, 2, c + u])
                    plsc.store_scatter(
                        obuf.at[p], [scat_rows, iota * 0 + (c + u)], m)
                return carry

            lax.fori_loop(0, n // 4, cbody, 0)

        def flush_out(t, p):
            for b in range(LANES):
                pltpu.async_copy(
                    obuf.at[p, b, pl.ds(0, CHB)],
                    out.at[t * LANES + b], osem[p])

        def wait_out(p):
            pltpu.make_async_copy(
                obuf.at[p, pl.ds(0, LANES), pl.ds(0, CHB)],
                out.at[pl.ds(0, LANES)], osem[p]).wait()

        def step(ci, p, do_wait_out, next_ci):
            wait_gather(p)
            if do_wait_out:
                wait_out(p)
            compute(p, CHB)
            flush_out(t0 + ci, p)
            if next_ci is not None:
                load_fire(t0 + next_ci, p)

        load_fire(t0, 0)
        load_fire(t0 + 1, 1)
        step(0, 0, False, 2)
        step(1, 1, False, 3)

        def pair(t, carry):
            ca = 2 * t
            step(ca, 0, True, ca + 2)
            step(ca + 1, 1, True, ca + 3)
            return carry

        lax.fori_loop(1, NCHUNK // 2 - 1, pair, 0)
        step(NCHUNK - 2, 0, True, None)
        step(NCHUNK - 1, 1, True, None)
        wait_out(0)
        wait_out(1)

        @pl.when(wid < n_extra)
        def _():
            te = NW * NCHUNK + wid
            load_fire(te, 0)
            wait_gather(0)
            compute(0, CHB)
            flush_out(te, 0)
            wait_out(0)

        if n_part:
            @pl.when(wid == n_extra)
            def _():
                tp = NW * NCHUNK + n_extra
                base = tp * CHB
                gsz = [GG] * (n_part // GG)
                if n_part % GG:
                    gsz.append(n_part % GG)
                for k in range(3):
                    pltpu.sync_copy(irefs[k].at[pl.ds(base, n_part)],
                                    idxv.at[0, k, pl.ds(0, n_part)])
                    pltpu.sync_copy(srefs[k].at[pl.ds(base, n_part)],
                                    sgnv.at[0, k, pl.ds(0, n_part)])

                def abody(g, carry):
                    o = g * LANES
                    for k in range(3):
                        ii = idxv[0, k, pl.ds(o, LANES)]
                        ss = sgnv[0, k, pl.ds(o, LANES)]
                        idxv[0, k, pl.ds(o, LANES)] = ii + jnp.where(
                            ss < 0.0, jnp.int32(NP), jnp.int32(0))
                    return carry

                lax.fori_loop(0, n_part // LANES, abody, 0)
                for k in range(3):
                    o = 0
                    for g in gsz:
                        pltpu.async_copy(
                            tbl.at[idxv.at[0, k, pl.ds(o, g)]],
                            gbuf.at[0, k, pl.ds(o, g)], gsem0)
                        o += g
                for k in range(3):
                    pltpu.make_async_copy(
                        tbl.at[pl.ds(0, n_part)],
                        gbuf.at[0, k, pl.ds(0, n_part)], gsem0).wait()

                def cbody(i, carry):
                    m = jnp.minimum(
                        jnp.minimum(gbuf[0, 0, i], gbuf[0, 1, i]),
                        gbuf[0, 2, i])
                    plsc.store_scatter(
                        obuf.at[0], [scat_rows, iota * 0 + i], m)
                    return carry

                lax.fori_loop(0, n_part, cbody, 0)
                flush_out(tp, 0)
                wait_out(0)

    return main


def _make_remap(M, n_tiles):
    assert n_tiles % KREMAP == 0

    def body(*refs):
        o_ref = refs[-1]
        o_ref[...] = jnp.concatenate([r[...] for r in refs[:-1]], axis=1)

    return pl.pallas_call(
        body,
        grid=(n_tiles // KREMAP,),
        in_specs=[
            pl.BlockSpec((LANES, CHB),
                         functools.partial(lambda j, i: (i * KREMAP + j, 0), j))
            for j in range(KREMAP)
        ],
        out_specs=pl.BlockSpec((LANES, KREMAP * CHB), lambda i: (0, i)),
        out_shape=jax.ShapeDtypeStruct((LANES, M), jnp.float32),
    )


def kernel(v, input_idx, input_sign):
    B, N = v.shape
    M, K = input_idx.shape
    assert B == LANES and K == 3

    NP = (N + NW * 8 - 1) // (NW * 8) * (NW * 8)
    RW = NP // NW

    n_full = M // CHB
    n_part = M - n_full * CHB
    n_extra = n_full - NW * NCHUNK
    assert 0 <= n_extra < NW
    n_tiles = n_full + (1 if n_part else 0)

    vt = jnp.zeros((NP, LANES), jnp.float32).at[:N].set(v.T)
    tbl = _make_table_builder(NP, RW)(vt)
    stack = _make_main(NP, M, n_tiles, n_extra, n_part)(
        tbl,
        input_idx[:, 0], input_idx[:, 1], input_idx[:, 2],
        input_sign[:, 0], input_sign[:, 1], input_sign[:, 2])
    return _make_remap(M, n_tiles)(*([stack] * KREMAP))


# speedup vs baseline: 2.1692x; 1.1717x over previous
import functools

import jax
import jax.numpy as jnp
from jax import lax
from jax.experimental import pallas as pl
from jax.experimental.pallas import tpu as pltpu
from jax.experimental.pallas import tpu_sc as plsc

NC = 2
NS = 16
NW = NC * NS
LANES = 16
CHB = 896
CHP = CHB + 1
GG = 112
NCHUNK = 14
KREMAP = 14


def _mesh():
    return plsc.VectorSubcoreMesh(
        core_axis_name="c", subcore_axis_name="s", num_cores=NC,
        num_subcores=NS)


def _params():
    return pltpu.CompilerParams(
        use_tc_tiling_on_sc=False, needs_layout_passes=False)


def _make_table_builder(N, NP, CW, CWL):
    SB = 784
    SBP = LANES + 1

    @functools.partial(
        pl.kernel,
        out_type=jax.ShapeDtypeStruct((2 * NP, LANES), jnp.float32),
        mesh=_mesh(),
        scratch_types=[
            pltpu.VMEM((LANES, SB), jnp.float32),
            pltpu.VMEM((SB, SBP), jnp.float32),
            pltpu.VMEM((SB, LANES), jnp.float32),
        ],
        compiler_params=_params(),
    )
    def build(v_hbm, tbl_hbm, vblk, vp, td):
        wid = lax.axis_index("c") * NS + lax.axis_index("s")
        iota = lax.iota(jnp.int32, LANES)

        def panel(c0, cw):
            pltpu.sync_copy(v_hbm.at[:, pl.ds(c0, cw)],
                            vblk.at[:, pl.ds(0, cw)])
            for b in range(LANES):
                colb = iota * 0 + b

                def tbody(g, carry):
                    o = g * LANES
                    plsc.store_scatter(vp, [o + iota, colb],
                                       vblk[b, pl.ds(o, LANES)])
                    return carry

                lax.fori_loop(0, cw // LANES, tbody, 0)

            def pa(i, carry):
                r = i * 4
                for u in range(4):
                    td[r + u] = 0.5 - 0.5 * vp[r + u, pl.ds(0, LANES)]
                return carry

            lax.fori_loop(0, cw // 4, pa, 0)
            pltpu.sync_copy(td.at[pl.ds(0, cw)], tbl_hbm.at[pl.ds(c0, cw)])

            def pb(i, carry):
                r = i * 4
                for u in range(4):
                    td[r + u] = 0.5 + 0.5 * vp[r + u, pl.ds(0, LANES)]
                return carry

            lax.fori_loop(0, cw // 4, pb, 0)
            pltpu.sync_copy(td.at[pl.ds(0, cw)],
                            tbl_hbm.at[pl.ds(NP + c0, cw)])

        def do(c0, cw):
            nfull = cw // SB
            for h in range(nfull):
                panel(c0 + h * SB, SB)
            if cw - nfull * SB:
                panel(c0 + nfull * SB, cw - nfull * SB)

        @pl.when(wid < NW - 1)
        def _():
            do(wid * CW, CW)

        @pl.when(wid == NW - 1)
        def _():
            do((NW - 1) * CW, CWL)

    return build


def _make_main(NP, M, n_tiles, n_extra, n_part):
    assert n_part % LANES == 0

    @functools.partial(
        pl.kernel,
        out_type=jax.ShapeDtypeStruct((n_tiles * LANES, CHB), jnp.float32),
        mesh=_mesh(),
        scratch_types=[
            pltpu.VMEM((2, 3, CHB), jnp.int32),
            pltpu.VMEM((2, 3, CHB), jnp.float32),
            pltpu.VMEM((2, 3, CHB, LANES), jnp.float32),
            pltpu.VMEM((2, LANES, CHP), jnp.float32),
            pltpu.SemaphoreType.DMA,
            pltpu.SemaphoreType.DMA,
            pltpu.SemaphoreType.DMA,
            pltpu.SemaphoreType.DMA,
        ],
        compiler_params=_params(),
    )
    def main(tbl, i0, i1, i2, s0, s1, s2, out, idxv, sgnv, gbuf, obuf,
             gsem0, gsem1, osem0, osem1):
        gsem = (gsem0, gsem1)
        osem = (osem0, osem1)
        irefs = (i0, i1, i2)
        srefs = (s0, s1, s2)
        wid = lax.axis_index("c") * NS + lax.axis_index("s")
        t0 = wid * NCHUNK
        iota = lax.iota(jnp.int32, LANES)
        scat_rows = iota * 0 + iota

        def adjust(p, n):
            def abody(g, carry):
                o = g * 64
                for k in range(3):
                    for u in range(4):
                        oo = o + u * LANES
                        ii = idxv[p, k, pl.ds(oo, LANES)]
                        ss = sgnv[p, k, pl.ds(oo, LANES)]
                        idxv[p, k, pl.ds(oo, LANES)] = ii + jnp.where(
                            ss < 0.0, jnp.int32(NP), jnp.int32(0))
                return carry

            lax.fori_loop(0, n // 64, abody, 0)

        def load_fire(t, p):
            base = t * CHB
            for k in range(3):
                pltpu.sync_copy(irefs[k].at[pl.ds(base, CHB)], idxv.at[p, k])
                pltpu.sync_copy(srefs[k].at[pl.ds(base, CHB)], sgnv.at[p, k])
            adjust(p, CHB)
            for k in range(3):
                for j in range(CHB // GG):
                    pltpu.async_copy(
                        tbl.at[idxv.at[p, k, pl.ds(j * GG, GG)]],
                        gbuf.at[p, k, pl.ds(j * GG, GG)],
                        gsem[p])

        def wait_gather(p):
            for k in range(3):
                pltpu.make_async_copy(
                    tbl.at[pl.ds(0, CHB)], gbuf.at[p, k], gsem[p]).wait()

        def compute(p, n):
            def cbody(i, carry):
                c = i * 4
                for u in range(4):
                    m = jnp.minimum(
                        jnp.minimum(gbuf[p, 0, c + u], gbuf[p, 1, c + u]),
                        gbuf[p, 2, c + u])
                    plsc.store_scatter(
                        obuf.at[p], [scat_rows, iota * 0 + (c + u)], m)
                return carry

            lax.fori_loop(0, n // 4, cbody, 0)

        def flush_out(t, p):
            for b in range(LANES):
                pltpu.async_copy(
                    obuf.at[p, b, pl.ds(0, CHB)],
                    out.at[t * LANES + b], osem[p])

        def wait_out(p):
            pltpu.make_async_copy(
                obuf.at[p, pl.ds(0, LANES), pl.ds(0, CHB)],
                out.at[pl.ds(0, LANES)], osem[p]).wait()

        def step(ci, p, do_wait_out, next_ci):
            wait_gather(p)
            if do_wait_out:
                wait_out(p)
            compute(p, CHB)
            flush_out(t0 + ci, p)
            if next_ci is not None:
                load_fire(t0 + next_ci, p)

        load_fire(t0, 0)
        load_fire(t0 + 1, 1)
        step(0, 0, False, 2)
        step(1, 1, False, 3)

        def pair(t, carry):
            ca = 2 * t
            step(ca, 0, True, ca + 2)
            step(ca + 1, 1, True, ca + 3)
            return carry

        lax.fori_loop(1, NCHUNK // 2 - 1, pair, 0)
        step(NCHUNK - 2, 0, True, None)
        step(NCHUNK - 1, 1, True, None)
        wait_out(0)
        wait_out(1)

        @pl.when(wid < n_extra)
        def _():
            te = NW * NCHUNK + wid
            load_fire(te, 0)
            wait_gather(0)
            compute(0, CHB)
            flush_out(te, 0)
            wait_out(0)

        if n_part:
            @pl.when(wid == n_extra)
            def _():
                tp = NW * NCHUNK + n_extra
                base = tp * CHB
                gsz = [GG] * (n_part // GG)
                if n_part % GG:
                    gsz.append(n_part % GG)
                for k in range(3):
                    pltpu.sync_copy(irefs[k].at[pl.ds(base, n_part)],
                                    idxv.at[0, k, pl.ds(0, n_part)])
                    pltpu.sync_copy(srefs[k].at[pl.ds(base, n_part)],
                                    sgnv.at[0, k, pl.ds(0, n_part)])

                def abody(g, carry):
                    o = g * LANES
                    for k in range(3):
                        ii = idxv[0, k, pl.ds(o, LANES)]
                        ss = sgnv[0, k, pl.ds(o, LANES)]
                        idxv[0, k, pl.ds(o, LANES)] = ii + jnp.where(
                            ss < 0.0, jnp.int32(NP), jnp.int32(0))
                    return carry

                lax.fori_loop(0, n_part // LANES, abody, 0)
                for k in range(3):
                    o = 0
                    for g in gsz:
                        pltpu.async_copy(
                            tbl.at[idxv.at[0, k, pl.ds(o, g)]],
                            gbuf.at[0, k, pl.ds(o, g)], gsem0)
                        o += g
                for k in range(3):
                    pltpu.make_async_copy(
                        tbl.at[pl.ds(0, n_part)],
                        gbuf.at[0, k, pl.ds(0, n_part)], gsem0).wait()

                def cbody(i, carry):
                    m = jnp.minimum(
                        jnp.minimum(gbuf[0, 0, i], gbuf[0, 1, i]),
                        gbuf[0, 2, i])
                    plsc.store_scatter(
                        obuf.at[0], [scat_rows, iota * 0 + i], m)
                    return carry

                lax.fori_loop(0, n_part, cbody, 0)
                flush_out(tp, 0)
                wait_out(0)

    return main


def _make_remap(M, n_tiles):
    assert n_tiles % KREMAP == 0

    def body(*refs):
        o_ref = refs[-1]
        o_ref[...] = jnp.concatenate([r[...] for r in refs[:-1]], axis=1)

    return pl.pallas_call(
        body,
        grid=(n_tiles // KREMAP,),
        in_specs=[
            pl.BlockSpec((LANES, CHB),
                         functools.partial(lambda j, i: (i * KREMAP + j, 0), j))
            for j in range(KREMAP)
        ],
        out_specs=pl.BlockSpec((LANES, KREMAP * CHB), lambda i: (0, i)),
        out_shape=jax.ShapeDtypeStruct((LANES, M), jnp.float32),
    )


def kernel(v, input_idx, input_sign):
    B, N = v.shape
    M, K = input_idx.shape
    assert B == LANES and K == 3

    CW = 3136
    CWL = N - (NW - 1) * CW
    assert 0 < CWL <= CW and CWL % LANES == 0
    NP = (N + 7) // 8 * 8

    n_full = M // CHB
    n_part = M - n_full * CHB
    n_extra = n_full - NW * NCHUNK
    assert 0 <= n_extra < NW
    n_tiles = n_full + (1 if n_part else 0)

    tbl = _make_table_builder(N, NP, CW, CWL)(v)
    stack = _make_main(NP, M, n_tiles, n_extra, n_part)(
        tbl,
        input_idx[:, 0], input_idx[:, 1], input_idx[:, 2],
        input_sign[:, 0], input_sign[:, 1], input_sign[:, 2])
    return _make_remap(M, n_tiles)(*([stack] * KREMAP))


# speedup vs baseline: 2.1744x; 1.0024x over previous
import functools

import jax
import jax.numpy as jnp
from jax import lax
from jax.experimental import pallas as pl
from jax.experimental.pallas import tpu as pltpu
from jax.experimental.pallas import tpu_sc as plsc

NC = 2
NS = 16
NW = NC * NS
LANES = 16
CHB = 896
CHP = CHB + 1
GG = 112
NCHUNK = 14
KREMAP = 14


def _mesh():
    return plsc.VectorSubcoreMesh(
        core_axis_name="c", subcore_axis_name="s", num_cores=NC,
        num_subcores=NS)


def _params():
    return pltpu.CompilerParams(
        use_tc_tiling_on_sc=False, needs_layout_passes=False)


def _make_table_builder(N, NP, CW, CWL):
    SB = 784
    SBP = LANES + 1

    @functools.partial(
        pl.kernel,
        out_type=jax.ShapeDtypeStruct((2 * NP, LANES), jnp.float32),
        mesh=_mesh(),
        scratch_types=[
            pltpu.VMEM((LANES, SB), jnp.float32),
            pltpu.VMEM((SB, SBP), jnp.float32),
            pltpu.VMEM((SB, LANES), jnp.float32),
        ],
        compiler_params=_params(),
    )
    def build(v_hbm, tbl_hbm, vblk, vp, td):
        wid = lax.axis_index("c") * NS + lax.axis_index("s")
        iota = lax.iota(jnp.int32, LANES)

        def panel(c0, cw):
            pltpu.sync_copy(v_hbm.at[:, pl.ds(c0, cw)],
                            vblk.at[:, pl.ds(0, cw)])
            for b in range(LANES):
                colb = iota * 0 + b

                def tbody(g, carry):
                    o = g * LANES
                    plsc.store_scatter(vp, [o + iota, colb],
                                       vblk[b, pl.ds(o, LANES)])
                    return carry

                lax.fori_loop(0, cw // LANES, tbody, 0)

            def pa(i, carry):
                r = i * 4
                for u in range(4):
                    td[r + u] = 0.5 - 0.5 * vp[r + u, pl.ds(0, LANES)]
                return carry

            lax.fori_loop(0, cw // 4, pa, 0)
            pltpu.sync_copy(td.at[pl.ds(0, cw)], tbl_hbm.at[pl.ds(c0, cw)])

            def pb(i, carry):
                r = i * 4
                for u in range(4):
                    td[r + u] = 0.5 + 0.5 * vp[r + u, pl.ds(0, LANES)]
                return carry

            lax.fori_loop(0, cw // 4, pb, 0)
            pltpu.sync_copy(td.at[pl.ds(0, cw)],
                            tbl_hbm.at[pl.ds(NP + c0, cw)])

        def do(c0, cw):
            nfull = cw // SB
            for h in range(nfull):
                panel(c0 + h * SB, SB)
            if cw - nfull * SB:
                panel(c0 + nfull * SB, cw - nfull * SB)

        @pl.when(wid < NW - 1)
        def _():
            do(wid * CW, CW)

        @pl.when(wid == NW - 1)
        def _():
            do((NW - 1) * CW, CWL)

    return build


def _make_main(NP, M, n_tiles, n_extra, n_part):
    assert n_part % LANES == 0

    @functools.partial(
        pl.kernel,
        out_type=jax.ShapeDtypeStruct((n_tiles * LANES, CHB), jnp.float32),
        mesh=_mesh(),
        scratch_types=[
            pltpu.VMEM((2, 3, CHB), jnp.int32),
            pltpu.VMEM((2, 3, CHB), jnp.float32),
            pltpu.VMEM((2, 3, CHB, LANES), jnp.float32),
            pltpu.VMEM((2, LANES, CHP), jnp.float32),
            pltpu.SemaphoreType.DMA,
            pltpu.SemaphoreType.DMA,
            pltpu.SemaphoreType.DMA,
            pltpu.SemaphoreType.DMA,
        ],
        compiler_params=_params(),
    )
    def main(tbl, i0, i1, i2, s0, s1, s2, out, idxv, sgnv, gbuf, obuf,
             gsem0, gsem1, osem0, osem1):
        gsem = (gsem0, gsem1)
        osem = (osem0, osem1)
        irefs = (i0, i1, i2)
        srefs = (s0, s1, s2)
        wid = lax.axis_index("c") * NS + lax.axis_index("s")
        t0 = wid * NCHUNK
        iota = lax.iota(jnp.int32, LANES)
        scat_rows = iota * 0 + iota

        def adjust(p, n):
            def abody(g, carry):
                o = g * 64
                for k in range(3):
                    for u in range(4):
                        oo = o + u * LANES
                        ii = idxv[p, k, pl.ds(oo, LANES)]
                        ss = sgnv[p, k, pl.ds(oo, LANES)]
                        idxv[p, k, pl.ds(oo, LANES)] = ii + jnp.where(
                            ss < 0.0, jnp.int32(NP), jnp.int32(0))
                return carry

            lax.fori_loop(0, n // 64, abody, 0)

        def load_fire(t, p):
            base = t * CHB
            for k in range(3):
                pltpu.sync_copy(irefs[k].at[pl.ds(base, CHB)], idxv.at[p, k])
                pltpu.sync_copy(srefs[k].at[pl.ds(base, CHB)], sgnv.at[p, k])
            adjust(p, CHB)
            for k in range(3):
                for j in range(CHB // GG):
                    pltpu.async_copy(
                        tbl.at[idxv.at[p, k, pl.ds(j * GG, GG)]],
                        gbuf.at[p, k, pl.ds(j * GG, GG)],
                        gsem[p])

        def wait_gather(p):
            for k in range(3):
                pltpu.make_async_copy(
                    tbl.at[pl.ds(0, CHB)], gbuf.at[p, k], gsem[p]).wait()

        def compute(p, n):
            def cbody(i, carry):
                c = i * 4
                for u in range(4):
                    m = jnp.minimum(
                        jnp.minimum(gbuf[p, 0, c + u], gbuf[p, 1, c + u]),
                        gbuf[p, 2, c + u])
                    plsc.store_scatter(
                        obuf.at[p], [scat_rows, iota * 0 + (c + u)], m)
                return carry

            lax.fori_loop(0, n // 4, cbody, 0)

        def flush_out(t, p):
            pltpu.async_copy(
                obuf.at[p, pl.ds(0, LANES), pl.ds(0, CHB)],
                out.at[pl.ds(t * LANES, LANES)], osem[p])

        def wait_out(p):
            pltpu.make_async_copy(
                obuf.at[p, pl.ds(0, LANES), pl.ds(0, CHB)],
                out.at[pl.ds(0, LANES)], osem[p]).wait()

        def step(ci, p, do_wait_out, next_ci):
            wait_gather(p)
            if do_wait_out:
                wait_out(p)
            compute(p, CHB)
            flush_out(t0 + ci, p)
            if next_ci is not None:
                load_fire(t0 + next_ci, p)

        load_fire(t0, 0)
        load_fire(t0 + 1, 1)
        step(0, 0, False, 2)
        step(1, 1, False, 3)

        def pair(t, carry):
            ca = 2 * t
            step(ca, 0, True, ca + 2)
            step(ca + 1, 1, True, ca + 3)
            return carry

        lax.fori_loop(1, NCHUNK // 2 - 1, pair, 0)
        step(NCHUNK - 2, 0, True, None)
        step(NCHUNK - 1, 1, True, None)
        wait_out(0)
        wait_out(1)

        @pl.when(wid < n_extra)
        def _():
            te = NW * NCHUNK + wid
            load_fire(te, 0)
            wait_gather(0)
            compute(0, CHB)
            flush_out(te, 0)
            wait_out(0)

        if n_part:
            @pl.when(wid == n_extra)
            def _():
                tp = NW * NCHUNK + n_extra
                base = tp * CHB
                gsz = [GG] * (n_part // GG)
                if n_part % GG:
                    gsz.append(n_part % GG)
                for k in range(3):
                    pltpu.sync_copy(irefs[k].at[pl.ds(base, n_part)],
                                    idxv.at[0, k, pl.ds(0, n_part)])
                    pltpu.sync_copy(srefs[k].at[pl.ds(base, n_part)],
                                    sgnv.at[0, k, pl.ds(0, n_part)])

                def abody(g, carry):
                    o = g * LANES
                    for k in range(3):
                        ii = idxv[0, k, pl.ds(o, LANES)]
                        ss = sgnv[0, k, pl.ds(o, LANES)]
                        idxv[0, k, pl.ds(o, LANES)] = ii + jnp.where(
                            ss < 0.0, jnp.int32(NP), jnp.int32(0))
                    return carry

                lax.fori_loop(0, n_part // LANES, abody, 0)
                for k in range(3):
                    o = 0
                    for g in gsz:
                        pltpu.async_copy(
                            tbl.at[idxv.at[0, k, pl.ds(o, g)]],
                            gbuf.at[0, k, pl.ds(o, g)], gsem0)
                        o += g
                for k in range(3):
                    pltpu.make_async_copy(
                        tbl.at[pl.ds(0, n_part)],
                        gbuf.at[0, k, pl.ds(0, n_part)], gsem0).wait()

                def cbody(i, carry):
                    m = jnp.minimum(
                        jnp.minimum(gbuf[0, 0, i], gbuf[0, 1, i]),
                        gbuf[0, 2, i])
                    plsc.store_scatter(
                        obuf.at[0], [scat_rows, iota * 0 + i], m)
                    return carry

                lax.fori_loop(0, n_part, cbody, 0)
                flush_out(tp, 0)
                wait_out(0)

    return main


def _make_remap(M, n_tiles):
    assert n_tiles % KREMAP == 0

    def body(*refs):
        o_ref = refs[-1]
        o_ref[...] = jnp.concatenate([r[...] for r in refs[:-1]], axis=1)

    return pl.pallas_call(
        body,
        grid=(n_tiles // KREMAP,),
        in_specs=[
            pl.BlockSpec((LANES, CHB),
                         functools.partial(lambda j, i: (i * KREMAP + j, 0), j))
            for j in range(KREMAP)
        ],
        out_specs=pl.BlockSpec((LANES, KREMAP * CHB), lambda i: (0, i)),
        out_shape=jax.ShapeDtypeStruct((LANES, M), jnp.float32),
    )


def kernel(v, input_idx, input_sign):
    B, N = v.shape
    M, K = input_idx.shape
    assert B == LANES and K == 3

    CW = 3136
    CWL = N - (NW - 1) * CW
    assert 0 < CWL <= CW and CWL % LANES == 0
    NP = (N + 7) // 8 * 8

    n_full = M // CHB
    n_part = M - n_full * CHB
    n_extra = n_full - NW * NCHUNK
    assert 0 <= n_extra < NW
    n_tiles = n_full + (1 if n_part else 0)

    tbl = _make_table_builder(N, NP, CW, CWL)(v)
    stack = _make_main(NP, M, n_tiles, n_extra, n_part)(
        tbl,
        input_idx[:, 0], input_idx[:, 1], input_idx[:, 2],
        input_sign[:, 0], input_sign[:, 1], input_sign[:, 2])
    return _make_remap(M, n_tiles)(*([stack] * KREMAP))


# speedup vs baseline: 2.1933x; 1.0087x over previous
import functools

import jax
import jax.numpy as jnp
from jax import lax
from jax.experimental import pallas as pl
from jax.experimental.pallas import tpu as pltpu
from jax.experimental.pallas import tpu_sc as plsc

NC = 2
NS = 16
NW = NC * NS
LANES = 16
CHB = 896
CHP = CHB + 1
GG = 112
NCHUNK = 14
KREMAP = 14


def _mesh():
    return plsc.VectorSubcoreMesh(
        core_axis_name="c", subcore_axis_name="s", num_cores=NC,
        num_subcores=NS)


def _params():
    return pltpu.CompilerParams(
        use_tc_tiling_on_sc=False, needs_layout_passes=False)


def _make_table_builder(N, NP, CW, CWL):
    SB = 784
    SBP = LANES + 1

    @functools.partial(
        pl.kernel,
        out_type=jax.ShapeDtypeStruct((2 * NP, LANES), jnp.float32),
        mesh=_mesh(),
        scratch_types=[
            pltpu.VMEM((LANES, SB), jnp.float32),
            pltpu.VMEM((SB, SBP), jnp.float32),
            pltpu.VMEM((SB, LANES), jnp.float32),
        ],
        compiler_params=_params(),
    )
    def build(v_hbm, tbl_hbm, vblk, vp, td):
        wid = lax.axis_index("c") * NS + lax.axis_index("s")
        iota = lax.iota(jnp.int32, LANES)

        def panel(c0, cw):
            pltpu.sync_copy(v_hbm.at[:, pl.ds(c0, cw)],
                            vblk.at[:, pl.ds(0, cw)])
            for b in range(LANES):
                colb = iota * 0 + b

                def tbody(g, carry):
                    o = g * LANES
                    plsc.store_scatter(vp, [o + iota, colb],
                                       vblk[b, pl.ds(o, LANES)])
                    return carry

                lax.fori_loop(0, cw // LANES, tbody, 0)

            def pa(i, carry):
                r = i * 4
                for u in range(4):
                    td[r + u] = 0.5 - 0.5 * vp[r + u, pl.ds(0, LANES)]
                return carry

            lax.fori_loop(0, cw // 4, pa, 0)
            pltpu.sync_copy(td.at[pl.ds(0, cw)], tbl_hbm.at[pl.ds(c0, cw)])

            def pb(i, carry):
                r = i * 4
                for u in range(4):
                    td[r + u] = 0.5 + 0.5 * vp[r + u, pl.ds(0, LANES)]
                return carry

            lax.fori_loop(0, cw // 4, pb, 0)
            pltpu.sync_copy(td.at[pl.ds(0, cw)],
                            tbl_hbm.at[pl.ds(NP + c0, cw)])

        def do(c0, cw):
            nfull = cw // SB
            for h in range(nfull):
                panel(c0 + h * SB, SB)
            if cw - nfull * SB:
                panel(c0 + nfull * SB, cw - nfull * SB)

        @pl.when(wid < NW - 1)
        def _():
            do(wid * CW, CW)

        @pl.when(wid == NW - 1)
        def _():
            do((NW - 1) * CW, CWL)

    return build


def _make_main(NP, M, n_tiles, n_extra, n_part):
    assert n_part % LANES == 0

    @functools.partial(
        pl.kernel,
        out_type=jax.ShapeDtypeStruct((n_tiles * LANES, CHB), jnp.float32),
        mesh=_mesh(),
        scratch_types=[
            pltpu.VMEM((2, 3, CHB), jnp.int32),
            pltpu.VMEM((2, 3, CHB), jnp.float32),
            pltpu.VMEM((2, 3, CHB, LANES), jnp.float32),
            pltpu.VMEM((2, LANES, CHP), jnp.float32),
            pltpu.SemaphoreType.DMA,
            pltpu.SemaphoreType.DMA,
            pltpu.SemaphoreType.DMA,
            pltpu.SemaphoreType.DMA,
        ],
        compiler_params=_params(),
    )
    def main(tbl, i0, i1, i2, s0, s1, s2, out, idxv, sgnv, gbuf, obuf,
             gsem0, gsem1, osem0, osem1):
        gsem = (gsem0, gsem1)
        osem = (osem0, osem1)
        irefs = (i0, i1, i2)
        srefs = (s0, s1, s2)
        wid = lax.axis_index("c") * NS + lax.axis_index("s")
        t0 = wid * NCHUNK
        iota = lax.iota(jnp.int32, LANES)
        scat_rows = iota * 0 + iota

        def adjust(p, n):
            def abody(g, carry):
                o = g * 64
                for k in range(3):
                    for u in range(4):
                        oo = o + u * LANES
                        ii = idxv[p, k, pl.ds(oo, LANES)]
                        ss = sgnv[p, k, pl.ds(oo, LANES)]
                        idxv[p, k, pl.ds(oo, LANES)] = ii + jnp.where(
                            ss < 0.0, jnp.int32(NP), jnp.int32(0))
                return carry

            lax.fori_loop(0, n // 64, abody, 0)

        def load_fire(t, p):
            base = t * CHB
            for k in range(3):
                pltpu.sync_copy(irefs[k].at[pl.ds(base, CHB)], idxv.at[p, k])
                pltpu.sync_copy(srefs[k].at[pl.ds(base, CHB)], sgnv.at[p, k])
            adjust(p, CHB)
            for k in range(3):
                for j in range(CHB // GG):
                    pltpu.async_copy(
                        tbl.at[idxv.at[p, k, pl.ds(j * GG, GG)]],
                        gbuf.at[p, k, pl.ds(j * GG, GG)],
                        gsem[p])

        def wait_gather(p):
            for k in range(3):
                pltpu.make_async_copy(
                    tbl.at[pl.ds(0, CHB)], gbuf.at[p, k], gsem[p]).wait()

        def compute(p, n):
            def cbody(i, carry):
                c = i * 8
                for u in range(8):
                    m = jnp.minimum(
                        jnp.minimum(gbuf[p, 0, c + u], gbuf[p, 1, c + u]),
                        gbuf[p, 2, c + u])
                    plsc.store_scatter(
                        obuf.at[p], [scat_rows, iota * 0 + (c + u)], m)
                return carry

            lax.fori_loop(0, n // 8, cbody, 0)

        def flush_out(t, p):
            pltpu.async_copy(
                obuf.at[p, pl.ds(0, LANES), pl.ds(0, CHB)],
                out.at[pl.ds(t * LANES, LANES)], osem[p])

        def wait_out(p):
            pltpu.make_async_copy(
                obuf.at[p, pl.ds(0, LANES), pl.ds(0, CHB)],
                out.at[pl.ds(0, LANES)], osem[p]).wait()

        def step(ci, p, do_wait_out, next_ci):
            wait_gather(p)
            if do_wait_out:
                wait_out(p)
            compute(p, CHB)
            flush_out(t0 + ci, p)
            if next_ci is not None:
                load_fire(t0 + next_ci, p)

        load_fire(t0, 0)
        load_fire(t0 + 1, 1)
        step(0, 0, False, 2)
        step(1, 1, False, 3)

        def pair(t, carry):
            ca = 2 * t
            step(ca, 0, True, ca + 2)
            step(ca + 1, 1, True, ca + 3)
            return carry

        lax.fori_loop(1, NCHUNK // 2 - 1, pair, 0)
        te = NW * NCHUNK + wid

        wait_gather(0)
        wait_out(0)
        compute(0, CHB)
        flush_out(t0 + NCHUNK - 2, 0)

        @pl.when(wid < n_extra)
        def _():
            load_fire(te, 0)

        step(NCHUNK - 1, 1, True, None)
        wait_out(1)

        @pl.when(wid < n_extra)
        def _():
            wait_gather(0)
            wait_out(0)
            compute(0, CHB)
            flush_out(te, 0)
            wait_out(0)

        @pl.when(wid >= n_extra)
        def _():
            wait_out(0)

        if n_part:
            @pl.when(wid == n_extra)
            def _():
                tp = NW * NCHUNK + n_extra
                base = tp * CHB
                gsz = [GG] * (n_part // GG)
                if n_part % GG:
                    gsz.append(n_part % GG)
                for k in range(3):
                    pltpu.sync_copy(irefs[k].at[pl.ds(base, n_part)],
                                    idxv.at[0, k, pl.ds(0, n_part)])
                    pltpu.sync_copy(srefs[k].at[pl.ds(base, n_part)],
                                    sgnv.at[0, k, pl.ds(0, n_part)])

                def abody(g, carry):
                    o = g * LANES
                    for k in range(3):
                        ii = idxv[0, k, pl.ds(o, LANES)]
                        ss = sgnv[0, k, pl.ds(o, LANES)]
                        idxv[0, k, pl.ds(o, LANES)] = ii + jnp.where(
                            ss < 0.0, jnp.int32(NP), jnp.int32(0))
                    return carry

                lax.fori_loop(0, n_part // LANES, abody, 0)
                for k in range(3):
                    o = 0
                    for g in gsz:
                        pltpu.async_copy(
                            tbl.at[idxv.at[0, k, pl.ds(o, g)]],
                            gbuf.at[0, k, pl.ds(o, g)], gsem0)
                        o += g
                for k in range(3):
                    pltpu.make_async_copy(
                        tbl.at[pl.ds(0, n_part)],
                        gbuf.at[0, k, pl.ds(0, n_part)], gsem0).wait()

                def cbody(i, carry):
                    m = jnp.minimum(
                        jnp.minimum(gbuf[0, 0, i], gbuf[0, 1, i]),
                        gbuf[0, 2, i])
                    plsc.store_scatter(
                        obuf.at[0], [scat_rows, iota * 0 + i], m)
                    return carry

                lax.fori_loop(0, n_part, cbody, 0)
                flush_out(tp, 0)
                wait_out(0)

    return main


def _make_remap(M, n_tiles):
    assert n_tiles % KREMAP == 0

    def body(*refs):
        o_ref = refs[-1]
        o_ref[...] = jnp.concatenate([r[...] for r in refs[:-1]], axis=1)

    return pl.pallas_call(
        body,
        grid=(n_tiles // KREMAP,),
        in_specs=[
            pl.BlockSpec((LANES, CHB),
                         functools.partial(lambda j, i: (i * KREMAP + j, 0), j))
            for j in range(KREMAP)
        ],
        out_specs=pl.BlockSpec((LANES, KREMAP * CHB), lambda i: (0, i)),
        out_shape=jax.ShapeDtypeStruct((LANES, M), jnp.float32),
    )


def kernel(v, input_idx, input_sign):
    B, N = v.shape
    M, K = input_idx.shape
    assert B == LANES and K == 3

    CW = 3136
    CWL = N - (NW - 1) * CW
    assert 0 < CWL <= CW and CWL % LANES == 0
    NP = (N + 7) // 8 * 8

    n_full = M // CHB
    n_part = M - n_full * CHB
    n_extra = n_full - NW * NCHUNK
    assert 0 <= n_extra < NW
    n_tiles = n_full + (1 if n_part else 0)

    tbl = _make_table_builder(N, NP, CW, CWL)(v)
    stack = _make_main(NP, M, n_tiles, n_extra, n_part)(
        tbl,
        input_idx[:, 0], input_idx[:, 1], input_idx[:, 2],
        input_sign[:, 0], input_sign[:, 1], input_sign[:, 2])
    return _make_remap(M, n_tiles)(*([stack] * KREMAP))


# speedup vs baseline: 2.2849x; 1.0418x over previous
import functools

import jax
import jax.numpy as jnp
from jax import lax
from jax.experimental import pallas as pl
from jax.experimental.pallas import tpu as pltpu
from jax.experimental.pallas import tpu_sc as plsc

NC = 2
NS = 16
NW = NC * NS
LANES = 16
CHB = 896
CHP = CHB + 1
GG = 112
NCHUNK = 14
KREMAP = 14


def _mesh():
    return plsc.VectorSubcoreMesh(
        core_axis_name="c", subcore_axis_name="s", num_cores=NC,
        num_subcores=NS)


def _params():
    return pltpu.CompilerParams(
        use_tc_tiling_on_sc=False, needs_layout_passes=False)


def _make_table_builder(N, NP, CW, CWL):
    SB = 784
    SBP = LANES + 1

    @functools.partial(
        pl.kernel,
        out_type=jax.ShapeDtypeStruct((2 * NP, LANES), jnp.float32),
        mesh=_mesh(),
        scratch_types=[
            pltpu.VMEM((LANES, SB), jnp.float32),
            pltpu.VMEM((SB, SBP), jnp.float32),
            pltpu.VMEM((SB, LANES), jnp.float32),
            pltpu.VMEM((SB, LANES), jnp.float32),
            pltpu.SemaphoreType.DMA,
        ],
        compiler_params=_params(),
    )
    def build(v_hbm, tbl_hbm, vblk, vp, ta, tb, sem):
        wid = lax.axis_index("c") * NS + lax.axis_index("s")
        iota = lax.iota(jnp.int32, LANES)

        def panel(c0, cw, drain_cw):
            pltpu.sync_copy(v_hbm.at[:, pl.ds(c0, cw)],
                            vblk.at[:, pl.ds(0, cw)])
            for b in range(LANES):
                colb = iota * 0 + b

                def tbody(g, carry):
                    o = g * 64
                    for u in range(4):
                        oo = o + u * LANES
                        plsc.store_scatter(vp, [oo + iota, colb],
                                           vblk[b, pl.ds(oo, LANES)])
                    return carry

                lax.fori_loop(0, cw // 64, tbody, 0)

            if drain_cw:
                for half in range(2):
                    pltpu.make_async_copy(
                        ta.at[pl.ds(0, drain_cw)],
                        tbl_hbm.at[pl.ds(0, drain_cw)], sem).wait()

            def pa(i, carry):
                r = i * 4
                for u in range(4):
                    x = vp[r + u, pl.ds(0, LANES)]
                    ta[r + u] = 0.5 - 0.5 * x
                    tb[r + u] = 0.5 + 0.5 * x
                return carry

            lax.fori_loop(0, cw // 4, pa, 0)
            pltpu.async_copy(ta.at[pl.ds(0, cw)],
                             tbl_hbm.at[pl.ds(c0, cw)], sem)
            pltpu.async_copy(tb.at[pl.ds(0, cw)],
                             tbl_hbm.at[pl.ds(NP + c0, cw)], sem)

        def do(c0, cw):
            sizes = [SB] * (cw // SB)
            if cw - (cw // SB) * SB:
                sizes.append(cw - (cw // SB) * SB)
            off = 0
            prev = 0
            for sz in sizes:
                panel(c0 + off, sz, prev)
                off += sz
                prev = sz
            for half in range(2):
                pltpu.make_async_copy(
                    ta.at[pl.ds(0, prev)],
                    tbl_hbm.at[pl.ds(0, prev)], sem).wait()

        @pl.when(wid < NW - 1)
        def _():
            do(wid * CW, CW)

        @pl.when(wid == NW - 1)
        def _():
            do((NW - 1) * CW, CWL)

    return build


def _make_main(NP, M, n_tiles, n_extra, n_part):
    assert n_part % LANES == 0

    @functools.partial(
        pl.kernel,
        out_type=jax.ShapeDtypeStruct((n_tiles * LANES, CHB), jnp.float32),
        mesh=_mesh(),
        scratch_types=[
            pltpu.VMEM((2, 3, CHB), jnp.int32),
            pltpu.VMEM((2, 3, CHB), jnp.float32),
            pltpu.VMEM((2, 3, CHB, LANES), jnp.float32),
            pltpu.VMEM((2, LANES, CHP), jnp.float32),
            pltpu.SemaphoreType.DMA,
            pltpu.SemaphoreType.DMA,
            pltpu.SemaphoreType.DMA,
            pltpu.SemaphoreType.DMA,
        ],
        compiler_params=_params(),
    )
    def main(tbl, i0, i1, i2, s0, s1, s2, out, idxv, sgnv, gbuf, obuf,
             gsem0, gsem1, osem0, osem1):
        gsem = (gsem0, gsem1)
        osem = (osem0, osem1)
        irefs = (i0, i1, i2)
        srefs = (s0, s1, s2)
        wid = lax.axis_index("c") * NS + lax.axis_index("s")
        t0 = wid * NCHUNK
        iota = lax.iota(jnp.int32, LANES)
        scat_rows = iota * 0 + iota

        def adjust(p, n):
            def abody(g, carry):
                o = g * 64
                for k in range(3):
                    for u in range(4):
                        oo = o + u * LANES
                        ii = idxv[p, k, pl.ds(oo, LANES)]
                        ss = sgnv[p, k, pl.ds(oo, LANES)]
                        idxv[p, k, pl.ds(oo, LANES)] = ii + jnp.where(
                            ss < 0.0, jnp.int32(NP), jnp.int32(0))
                return carry

            lax.fori_loop(0, n // 64, abody, 0)

        def load_fire(t, p):
            base = t * CHB
            for k in range(3):
                pltpu.sync_copy(irefs[k].at[pl.ds(base, CHB)], idxv.at[p, k])
                pltpu.sync_copy(srefs[k].at[pl.ds(base, CHB)], sgnv.at[p, k])
            adjust(p, CHB)
            for k in range(3):
                for j in range(CHB // GG):
                    pltpu.async_copy(
                        tbl.at[idxv.at[p, k, pl.ds(j * GG, GG)]],
                        gbuf.at[p, k, pl.ds(j * GG, GG)],
                        gsem[p])

        def wait_gather(p):
            for k in range(3):
                pltpu.make_async_copy(
                    tbl.at[pl.ds(0, CHB)], gbuf.at[p, k], gsem[p]).wait()

        def compute(p, n):
            def cbody(i, carry):
                c = i * 8
                for u in range(8):
                    m = jnp.minimum(
                        jnp.minimum(gbuf[p, 0, c + u], gbuf[p, 1, c + u]),
                        gbuf[p, 2, c + u])
                    plsc.store_scatter(
                        obuf.at[p], [scat_rows, iota * 0 + (c + u)], m)
                return carry

            lax.fori_loop(0, n // 8, cbody, 0)

        def flush_out(t, p):
            pltpu.async_copy(
                obuf.at[p, pl.ds(0, LANES), pl.ds(0, CHB)],
                out.at[pl.ds(t * LANES, LANES)], osem[p])

        def wait_out(p):
            pltpu.make_async_copy(
                obuf.at[p, pl.ds(0, LANES), pl.ds(0, CHB)],
                out.at[pl.ds(0, LANES)], osem[p]).wait()

        def step(ci, p, do_wait_out, next_ci):
            wait_gather(p)
            if do_wait_out:
                wait_out(p)
            compute(p, CHB)
            flush_out(t0 + ci, p)
            if next_ci is not None:
                load_fire(t0 + next_ci, p)

        load_fire(t0, 0)
        load_fire(t0 + 1, 1)
        step(0, 0, False, 2)
        step(1, 1, False, 3)

        def pair(t, carry):
            ca = 2 * t
            step(ca, 0, True, ca + 2)
            step(ca + 1, 1, True, ca + 3)
            return carry

        lax.fori_loop(1, NCHUNK // 2 - 1, pair, 0)
        te = NW * NCHUNK + wid

        wait_gather(0)
        wait_out(0)
        compute(0, CHB)
        flush_out(t0 + NCHUNK - 2, 0)

        @pl.when(wid < n_extra)
        def _():
            load_fire(te, 0)

        step(NCHUNK - 1, 1, True, None)
        wait_out(1)

        @pl.when(wid < n_extra)
        def _():
            wait_gather(0)
            wait_out(0)
            compute(0, CHB)
            flush_out(te, 0)
            wait_out(0)

        @pl.when(wid >= n_extra)
        def _():
            wait_out(0)

        if n_part:
            @pl.when(wid == n_extra)
            def _():
                tp = NW * NCHUNK + n_extra
                base = tp * CHB
                gsz = [GG] * (n_part // GG)
                if n_part % GG:
                    gsz.append(n_part % GG)
                for k in range(3):
                    pltpu.sync_copy(irefs[k].at[pl.ds(base, n_part)],
                                    idxv.at[0, k, pl.ds(0, n_part)])
                    pltpu.sync_copy(srefs[k].at[pl.ds(base, n_part)],
                                    sgnv.at[0, k, pl.ds(0, n_part)])

                def abody(g, carry):
                    o = g * LANES
                    for k in range(3):
                        ii = idxv[0, k, pl.ds(o, LANES)]
                        ss = sgnv[0, k, pl.ds(o, LANES)]
                        idxv[0, k, pl.ds(o, LANES)] = ii + jnp.where(
                            ss < 0.0, jnp.int32(NP), jnp.int32(0))
                    return carry

                lax.fori_loop(0, n_part // LANES, abody, 0)
                for k in range(3):
                    o = 0
                    for g in gsz:
                        pltpu.async_copy(
                            tbl.at[idxv.at[0, k, pl.ds(o, g)]],
                            gbuf.at[0, k, pl.ds(o, g)], gsem0)
                        o += g
                for k in range(3):
                    pltpu.make_async_copy(
                        tbl.at[pl.ds(0, n_part)],
                        gbuf.at[0, k, pl.ds(0, n_part)], gsem0).wait()

                def cbody(i, carry):
                    m = jnp.minimum(
                        jnp.minimum(gbuf[0, 0, i], gbuf[0, 1, i]),
                        gbuf[0, 2, i])
                    plsc.store_scatter(
                        obuf.at[0], [scat_rows, iota * 0 + i], m)
                    return carry

                lax.fori_loop(0, n_part, cbody, 0)
                flush_out(tp, 0)
                wait_out(0)

    return main


def _make_remap(M, n_tiles):
    assert n_tiles % KREMAP == 0

    def body(*refs):
        o_ref = refs[-1]
        o_ref[...] = jnp.concatenate([r[...] for r in refs[:-1]], axis=1)

    return pl.pallas_call(
        body,
        grid=(n_tiles // KREMAP,),
        in_specs=[
            pl.BlockSpec((LANES, CHB),
                         functools.partial(lambda j, i: (i * KREMAP + j, 0), j))
            for j in range(KREMAP)
        ],
        out_specs=pl.BlockSpec((LANES, KREMAP * CHB), lambda i: (0, i)),
        out_shape=jax.ShapeDtypeStruct((LANES, M), jnp.float32),
    )


def kernel(v, input_idx, input_sign):
    B, N = v.shape
    M, K = input_idx.shape
    assert B == LANES and K == 3

    CW = 3136
    CWL = N - (NW - 1) * CW
    assert 0 < CWL <= CW and CWL % LANES == 0
    NP = (N + 7) // 8 * 8

    n_full = M // CHB
    n_part = M - n_full * CHB
    n_extra = n_full - NW * NCHUNK
    assert 0 <= n_extra < NW
    n_tiles = n_full + (1 if n_part else 0)

    tbl = _make_table_builder(N, NP, CW, CWL)(v)
    stack = _make_main(NP, M, n_tiles, n_extra, n_part)(
        tbl,
        input_idx[:, 0], input_idx[:, 1], input_idx[:, 2],
        input_sign[:, 0], input_sign[:, 1], input_sign[:, 2])
    return _make_remap(M, n_tiles)(*([stack] * KREMAP))
